# Initial kernel scaffold; baseline (speedup 1.0000x reference)
import jax
import jax.numpy as jnp
from jax.experimental import pallas as pl


def kernel(x, edge_attr, params, edge_index):
    raise NotImplementedError("write your pallas kernel here")


# speedup vs baseline: 2.7817x; 2.7817x over previous
import functools

import jax
import jax.numpy as jnp
from jax import lax
from jax.experimental import pallas as pl
from jax.experimental.pallas import tpu as pltpu
from jax.experimental.pallas import tpu_sc as plsc

_EPS = 1e-5
_DTC = 0.01


def _ln(y, g, b):
    mu = jnp.mean(y, axis=-1, keepdims=True)
    var = jnp.mean((y - mu) ** 2, axis=-1, keepdims=True)
    return (y - mu) * lax.rsqrt(var + _EPS) * g + b


def _w(shape):
    return pl.BlockSpec(shape, lambda i: tuple(0 for _ in shape))


def _row(block_rows, cols):
    return pl.BlockSpec((block_rows, cols), lambda i: (i, 0))


def _tc_node_encoder(x, mp, lnp, wa, wb):
    n = x.shape[0]
    bn = 2000

    def body(x_ref, w1_ref, b1_ref, w2_ref, b2_ref, g_ref, gb_ref,
             wa_ref, wb_ref, hn_ref, a_ref, b_ref, m_ref):
        xx = x_ref[...]
        h = jnp.maximum(
            jnp.dot(xx, w1_ref[...], preferred_element_type=jnp.float32)
            + b1_ref[...], 0.0)
        y = jnp.dot(h, w2_ref[...], preferred_element_type=jnp.float32) + b2_ref[...]
        hn = _ln(y, g_ref[...], gb_ref[...])
        hn_ref[...] = hn
        a_ref[...] = jnp.dot(hn, wa_ref[...], preferred_element_type=jnp.float32)
        b_ref[...] = jnp.dot(hn, wb_ref[...], preferred_element_type=jnp.float32)
        m_ref[...] = (xx[:, 1:2] <= xx[:, 2:3] + _DTC).astype(jnp.float32)

    return pl.pallas_call(
        body,
        grid=(n // bn,),
        in_specs=[_row(bn, 128), _w((128, 32)), _w((1, 32)), _w((32, 32)),
                  _w((1, 32)), _w((1, 32)), _w((1, 32)), _w((32, 32)),
                  _w((32, 32))],
        out_specs=[_row(bn, 32), _row(bn, 32), _row(bn, 32), _row(bn, 1)],
        out_shape=[jax.ShapeDtypeStruct((n, 32), jnp.float32),
                   jax.ShapeDtypeStruct((n, 32), jnp.float32),
                   jax.ShapeDtypeStruct((n, 32), jnp.float32),
                   jax.ShapeDtypeStruct((n, 1), jnp.float32)],
    )(x, mp["w1"], mp["b1"].reshape(1, 32), mp["w2"], mp["b2"].reshape(1, 32),
      lnp["g"].reshape(1, 32), lnp["b"].reshape(1, 32), wa, wb)


def _tc_edge_encoder(ea, mp, lnp):
    e = ea.shape[0]
    be = 3200

    def body(ea_ref, w1_ref, b1_ref, w2_ref, b2_ref, g_ref, gb_ref, out_ref):
        aa = ea_ref[...]
        h = jnp.maximum(
            jnp.dot(aa, w1_ref[...], preferred_element_type=jnp.float32)
            + b1_ref[...], 0.0)
        y = jnp.dot(h, w2_ref[...], preferred_element_type=jnp.float32) + b2_ref[...]
        out_ref[...] = _ln(y, g_ref[...], gb_ref[...])

    return pl.pallas_call(
        body,
        grid=(e // be,),
        in_specs=[_row(be, 2), _w((2, 32)), _w((1, 32)), _w((32, 32)),
                  _w((1, 32)), _w((1, 32)), _w((1, 32))],
        out_specs=_row(be, 32),
        out_shape=jax.ShapeDtypeStruct((e, 32), jnp.float32),
    )(ea, mp["w1"], mp["b1"].reshape(1, 32), mp["w2"], mp["b2"].reshape(1, 32),
      lnp["g"].reshape(1, 32), lnp["b"].reshape(1, 32))


def _tc_edge_update(he, ga, gb, mp, lnp):
    e = he.shape[0]
    be = 3200

    def body(he_ref, ga_ref, gb_ref, w1_ref, b1_ref, w2_ref, b2_ref,
             g_ref, gb2_ref, out_ref):
        hh = he_ref[...]
        pre = (jnp.dot(hh, w1_ref[...], preferred_element_type=jnp.float32)
               + ga_ref[...] + gb_ref[...] + b1_ref[...])
        h = jnp.maximum(pre, 0.0)
        y = jnp.dot(h, w2_ref[...], preferred_element_type=jnp.float32) + b2_ref[...]
        out_ref[...] = hh + _ln(y, g_ref[...], gb2_ref[...])

    return pl.pallas_call(
        body,
        grid=(e // be,),
        in_specs=[_row(be, 32)] * 3 + [_w((32, 32)), _w((1, 32)), _w((32, 32)),
                                       _w((1, 32)), _w((1, 32)), _w((1, 32))],
        out_specs=_row(be, 32),
        out_shape=jax.ShapeDtypeStruct((e, 32), jnp.float32),
    )(he, ga, gb, mp["w1"][0:32], mp["b1"].reshape(1, 32), mp["w2"],
      mp["b2"].reshape(1, 32), lnp["g"].reshape(1, 32), lnp["b"].reshape(1, 32))


def _tc_node_update(hn, p0, p1, mp, lnp, wa=None, wb=None):
    n = hn.shape[0]
    bn = 2000
    emit_ab = wa is not None

    def body(hn_ref, p0_ref, p1_ref, w1a_ref, w1b_ref, b1_ref, w2_ref, b2_ref,
             g_ref, gb_ref, *rest):
        if emit_ab:
            wa_ref, wb_ref, out_ref, a_ref, b_ref = rest
        else:
            (out_ref,) = rest
        hh = hn_ref[...]
        aggr = p0_ref[...] + p1_ref[...]
        pre = (jnp.dot(hh, w1a_ref[...], preferred_element_type=jnp.float32)
               + jnp.dot(aggr, w1b_ref[...], preferred_element_type=jnp.float32)
               + b1_ref[...])
        h = jnp.maximum(pre, 0.0)
        y = jnp.dot(h, w2_ref[...], preferred_element_type=jnp.float32) + b2_ref[...]
        hn_new = hh + _ln(y, g_ref[...], gb_ref[...])
        out_ref[...] = hn_new
        if emit_ab:
            a_ref[...] = jnp.dot(hn_new, wa_ref[...], preferred_element_type=jnp.float32)
            b_ref[...] = jnp.dot(hn_new, wb_ref[...], preferred_element_type=jnp.float32)

    in_specs = [_row(bn, 32)] * 3 + [_w((32, 32)), _w((32, 32)), _w((1, 32)),
                                     _w((32, 32)), _w((1, 32)), _w((1, 32)),
                                     _w((1, 32))]
    args = [hn, p0, p1, mp["w1"][0:32], mp["w1"][32:64],
            mp["b1"].reshape(1, 32), mp["w2"], mp["b2"].reshape(1, 32),
            lnp["g"].reshape(1, 32), lnp["b"].reshape(1, 32)]
    if emit_ab:
        in_specs += [_w((32, 32)), _w((32, 32))]
        args += [wa, wb]
        out_specs = [_row(bn, 32)] * 3
        out_shape = [jax.ShapeDtypeStruct((n, 32), jnp.float32)] * 3
    else:
        out_specs = _row(bn, 32)
        out_shape = jax.ShapeDtypeStruct((n, 32), jnp.float32)
    return pl.pallas_call(
        body, grid=(n // bn,), in_specs=in_specs, out_specs=out_specs,
        out_shape=out_shape)(*args)


def _tc_decoder(hn, m, mp):
    n = hn.shape[0]
    bn = 2000

    def body(hn_ref, m_ref, w1_ref, b1_ref, w2_ref, b2_ref, out_ref):
        h = jnp.maximum(
            jnp.dot(hn_ref[...], w1_ref[...], preferred_element_type=jnp.float32)
            + b1_ref[...], 0.0)
        y = jnp.dot(h, w2_ref[...], preferred_element_type=jnp.float32) + b2_ref[...]
        out_ref[...] = y * m_ref[...]

    return pl.pallas_call(
        body,
        grid=(n // bn,),
        in_specs=[_row(bn, 32), _row(bn, 1), _w((32, 32)), _w((1, 32)),
                  _w((32, 3)), _w((1, 3))],
        out_specs=_row(bn, 3),
        out_shape=jax.ShapeDtypeStruct((n, 3), jnp.float32),
    )(hn, m, mp["w1"], mp["b1"].reshape(1, 32), mp["w2"], mp["b2"].reshape(1, 3))


@functools.cache
def _mesh():
    return plsc.VectorSubcoreMesh(core_axis_name="c", subcore_axis_name="s")


_NOTILE = pltpu.CompilerParams(use_tc_tiling_on_sc=False)
_CHR = 8
_CH = _CHR * 128


def _sc_gather(a, b, src2, dst2):
    e = src2.shape[0] * 128
    nch = e // _CH
    tail = (e - nch * _CH) // 128

    @functools.partial(
        pl.kernel,
        mesh=_mesh(),
        out_type=[jax.ShapeDtypeStruct((e, 32), jnp.float32),
                  jax.ShapeDtypeStruct((e, 32), jnp.float32)],
        scratch_types=[pltpu.VMEM((_CHR, 128), jnp.int32),
                       pltpu.VMEM((_CHR, 128), jnp.int32),
                       pltpu.VMEM((_CH, 32), jnp.float32),
                       pltpu.VMEM((_CH, 32), jnp.float32),
                       pltpu.SemaphoreType.DMA,
                       pltpu.SemaphoreType.DMA],
        compiler_params=_NOTILE,
    )
    def k(a_hbm, b_hbm, s_hbm, d_hbm, ga_hbm, gb_hbm, si, di, ba, bb, sa, sb):
        c = lax.axis_index("c")
        s = lax.axis_index("s")
        wid = c * 16 + s
        trips = (nch - wid + 31) // 32

        def do_chunk(ch, rows):
            pltpu.sync_copy(s_hbm.at[pl.ds(ch * _CHR, rows)],
                            si.at[pl.ds(0, rows)])
            pltpu.sync_copy(d_hbm.at[pl.ds(ch * _CHR, rows)],
                            di.at[pl.ds(0, rows)])
            cps = []
            for j in range(rows):
                cps.append(pltpu.async_copy(
                    a_hbm.at[si.at[j]], ba.at[pl.ds(j * 128, 128)], sa))
                cps.append(pltpu.async_copy(
                    b_hbm.at[di.at[j]], bb.at[pl.ds(j * 128, 128)], sb))
            for cp in cps:
                cp.wait()
            pltpu.sync_copy(ba.at[pl.ds(0, rows * 128)],
                            ga_hbm.at[pl.ds(ch * _CH, rows * 128)])
            pltpu.sync_copy(bb.at[pl.ds(0, rows * 128)],
                            gb_hbm.at[pl.ds(ch * _CH, rows * 128)])

        def body(i, carry):
            do_chunk(wid + i * 32, _CHR)
            return carry

        lax.fori_loop(0, trips, body, 0)
        if tail:
            @pl.when(wid == 31)
            def _():
                do_chunk(nch, tail)

    return k(a, b, src2, dst2)


def _sc_scatter(he, dst2, zeros):
    n = zeros.shape[0]
    e = he.shape[0]
    nch = e // _CH
    tail = (e - nch * _CH) // 128
    per = n // 16

    @functools.partial(
        pl.kernel,
        mesh=_mesh(),
        out_type=[jax.ShapeDtypeStruct((n, 32), jnp.float32),
                  jax.ShapeDtypeStruct((n, 32), jnp.float32)],
        scratch_types=[pltpu.VMEM((_CHR, 128), jnp.int32),
                       pltpu.VMEM((_CH, 32), jnp.float32),
                       pltpu.VMEM_SHARED((n, 32), jnp.float32)],
        compiler_params=_NOTILE,
    )
    def k(he_hbm, d_hbm, z_hbm, o0, o1, di, be, acc):
        c = lax.axis_index("c")
        s = lax.axis_index("s")
        wid = c * 16 + s
        pltpu.sync_copy(z_hbm.at[pl.ds(s * per, per)], acc.at[pl.ds(s * per, per)])
        plsc.subcore_barrier()
        trips = (nch - wid + 31) // 32

        def do_chunk(ch, rows):
            pltpu.sync_copy(d_hbm.at[pl.ds(ch * _CHR, rows)],
                            di.at[pl.ds(0, rows)])
            pltpu.sync_copy(he_hbm.at[pl.ds(ch * _CH, rows * 128)],
                            be.at[pl.ds(0, rows * 128)])
            for j in range(rows):
                pltpu.sync_copy(be.at[pl.ds(j * 128, 128)], acc.at[di.at[j]],
                                add=True)

        def body(i, carry):
            do_chunk(wid + i * 32, _CHR)
            return carry

        lax.fori_loop(0, trips, body, 0)
        if tail:
            @pl.when(wid == 30)
            def _():
                do_chunk(nch, tail)
        plsc.subcore_barrier()

        @pl.when(c == 0)
        def _():
            pltpu.sync_copy(acc.at[pl.ds(s * per, per)], o0.at[pl.ds(s * per, per)])

        @pl.when(c == 1)
        def _():
            pltpu.sync_copy(acc.at[pl.ds(s * per, per)], o1.at[pl.ds(s * per, per)])

    return k(he, dst2, zeros)


def kernel(x, edge_attr, params, edge_index):
    src2 = edge_index[0].reshape(-1, 128)
    dst2 = edge_index[1].reshape(-1, 128)
    layers = params["layers"]
    ew = [lp["edge"]["w1"] for lp in layers]

    hn, a, b, m = _tc_node_encoder(x, params["enc_n"], params["enc_n_ln"],
                                   ew[0][32:64], ew[0][64:96])
    he = _tc_edge_encoder(edge_attr, params["enc_e"], params["enc_e_ln"])
    zeros = jnp.zeros((x.shape[0], 32), jnp.float32)

    for l, lp in enumerate(layers):
        ga, gb = _sc_gather(a, b, src2, dst2)
        he = _tc_edge_update(he, ga, gb, lp["edge"], lp["edge_ln"])
        p0, p1 = _sc_scatter(he, dst2, zeros)
        if l + 1 < len(layers):
            hn, a, b = _tc_node_update(hn, p0, p1, lp["node"], lp["node_ln"],
                                       ew[l + 1][32:64], ew[l + 1][64:96])
        else:
            hn = _tc_node_update(hn, p0, p1, lp["node"], lp["node_ln"])

    return _tc_decoder(hn, m, params["dec"])


# speedup vs baseline: 7.3960x; 2.6589x over previous
import functools

import jax
import jax.numpy as jnp
from jax import lax
from jax.experimental import pallas as pl
from jax.experimental.pallas import tpu as pltpu
from jax.experimental.pallas import tpu_sc as plsc

_EPS = 1e-5
_DTC = 0.01


def _ln(y, g, b):
    mu = jnp.mean(y, axis=-1, keepdims=True)
    var = jnp.mean((y - mu) ** 2, axis=-1, keepdims=True)
    return (y - mu) * lax.rsqrt(var + _EPS) * g + b


def _w(shape):
    return pl.BlockSpec(shape, lambda i: tuple(0 for _ in shape))


def _row(block_rows, cols):
    return pl.BlockSpec((block_rows, cols), lambda i: (i, 0))


def _kron4(w):
    return jnp.kron(jnp.eye(4, dtype=jnp.float32), w)


def _t4(v):
    return jnp.tile(v, 4).reshape(1, -1)


def _mavg():
    return jnp.kron(jnp.eye(4, dtype=jnp.float32),
                    jnp.full((32, 32), 1.0 / 32.0, jnp.float32))


def _pln(y, mavg, g, b):
    mu = jnp.dot(y, mavg, preferred_element_type=jnp.float32)
    d = y - mu
    var = jnp.dot(d * d, mavg, preferred_element_type=jnp.float32)
    return d * lax.rsqrt(var + _EPS) * g + b


def _dot(a, b):
    return jnp.dot(a, b, preferred_element_type=jnp.float32)


def _tc_node_encoder(x_p, mp, lnp, wa, wb):
    r = x_p.shape[0]

    def body(x_ref, w1_ref, b1_ref, w2_ref, b2_ref, g_ref, gb_ref, mavg_ref,
             wa_ref, wb_ref, hn_ref, a_ref, b_ref, m_ref):
        xx = x_ref[...]
        h = jnp.maximum(_dot(xx, w1_ref[...]) + b1_ref[...], 0.0)
        y = _dot(h, w2_ref[...]) + b2_ref[...]
        hn = _pln(y, mavg_ref[...], g_ref[...], gb_ref[...])
        hn_ref[...] = hn
        a_ref[...] = _dot(hn, wa_ref[...])
        b_ref[...] = _dot(hn, wb_ref[...])
        cols = []
        for gidx in range(4):
            z0 = xx[:, 128 * gidx + 1:128 * gidx + 2]
            t1 = xx[:, 128 * gidx + 2:128 * gidx + 3] + _DTC
            mg = (z0 <= t1).astype(jnp.float32)
            cols += [mg, mg, mg]
        m_ref[...] = jnp.concatenate(cols, axis=1)

    return pl.pallas_call(
        body,
        grid=(1,),
        in_specs=[_w((r, 512)), _w((512, 128)), _w((1, 128)), _w((128, 128)),
                  _w((1, 128)), _w((1, 128)), _w((1, 128)), _w((128, 128)),
                  _w((128, 128)), _w((128, 128))],
        out_specs=[_w((r, 128)), _w((r, 128)), _w((r, 128)), _w((r, 12))],
        out_shape=[jax.ShapeDtypeStruct((r, 128), jnp.float32),
                   jax.ShapeDtypeStruct((r, 128), jnp.float32),
                   jax.ShapeDtypeStruct((r, 128), jnp.float32),
                   jax.ShapeDtypeStruct((r, 12), jnp.float32)],
    )(x_p, _kron4(mp["w1"]), _t4(mp["b1"]), _kron4(mp["w2"]), _t4(mp["b2"]),
      _t4(lnp["g"]), _t4(lnp["b"]), _mavg(), _kron4(wa), _kron4(wb))


def _tc_edge_encoder(ea_v, mp, lnp):
    r1 = ea_v.shape[0]
    w1big = jnp.kron(jnp.eye(64, dtype=jnp.float32), mp["w1"])
    b1big = jnp.tile(mp["b1"], 64).reshape(1, -1)

    def body1(ea_ref, w_ref, b_ref, out_ref):
        out_ref[...] = jnp.maximum(_dot(ea_ref[...], w_ref[...]) + b_ref[...],
                                   0.0)

    be1 = 1000
    h = pl.pallas_call(
        body1,
        grid=(r1 // be1,),
        in_specs=[_row(be1, 128), _w((128, 2048)), _w((1, 2048))],
        out_specs=_row(be1, 2048),
        out_shape=jax.ShapeDtypeStruct((r1, 2048), jnp.float32),
    )(ea_v, w1big, b1big)

    hp = h.reshape(r1 * 16, 128)
    r = hp.shape[0]
    be = 2000

    def body2(h_ref, w2_ref, b2_ref, g_ref, gb_ref, mavg_ref, out_ref):
        y = _dot(h_ref[...], w2_ref[...]) + b2_ref[...]
        out_ref[...] = _pln(y, mavg_ref[...], g_ref[...], gb_ref[...])

    return pl.pallas_call(
        body2,
        grid=(r // be,),
        in_specs=[_row(be, 128), _w((128, 128)), _w((1, 128)), _w((1, 128)),
                  _w((1, 128)), _w((128, 128))],
        out_specs=_row(be, 128),
        out_shape=jax.ShapeDtypeStruct((r, 128), jnp.float32),
    )(hp, _kron4(mp["w2"]), _t4(mp["b2"]), _t4(lnp["g"]), _t4(lnp["b"]),
      _mavg())


def _tc_edge_update(he_p, ga_p, gb_p, mp, lnp):
    r = he_p.shape[0]
    be = 2000

    def body(he_ref, ga_ref, gb_ref, w1_ref, b1_ref, w2_ref, b2_ref,
             g_ref, gb2_ref, mavg_ref, out_ref):
        hh = he_ref[...]
        pre = _dot(hh, w1_ref[...]) + ga_ref[...] + gb_ref[...] + b1_ref[...]
        h = jnp.maximum(pre, 0.0)
        y = _dot(h, w2_ref[...]) + b2_ref[...]
        out_ref[...] = hh + _pln(y, mavg_ref[...], g_ref[...], gb2_ref[...])

    return pl.pallas_call(
        body,
        grid=(r // be,),
        in_specs=[_row(be, 128)] * 3 + [_w((128, 128)), _w((1, 128)),
                                        _w((128, 128)), _w((1, 128)),
                                        _w((1, 128)), _w((1, 128)),
                                        _w((128, 128))],
        out_specs=_row(be, 128),
        out_shape=jax.ShapeDtypeStruct((r, 128), jnp.float32),
    )(he_p, ga_p, gb_p, _kron4(mp["w1"][0:32]), _t4(mp["b1"]),
      _kron4(mp["w2"]), _t4(mp["b2"]), _t4(lnp["g"]), _t4(lnp["b"]), _mavg())


def _tc_node_update(hn_p, p0_p, p1_p, mp, lnp, wa=None, wb=None):
    r = hn_p.shape[0]
    emit_ab = wa is not None

    def body(hn_ref, p0_ref, p1_ref, w1a_ref, w1b_ref, b1_ref, w2_ref, b2_ref,
             g_ref, gb_ref, mavg_ref, *rest):
        if emit_ab:
            wa_ref, wb_ref, out_ref, a_ref, b_ref = rest
        else:
            (out_ref,) = rest
        hh = hn_ref[...]
        aggr = p0_ref[...] + p1_ref[...]
        pre = (_dot(hh, w1a_ref[...]) + _dot(aggr, w1b_ref[...]) + b1_ref[...])
        h = jnp.maximum(pre, 0.0)
        y = _dot(h, w2_ref[...]) + b2_ref[...]
        hn_new = hh + _pln(y, mavg_ref[...], g_ref[...], gb_ref[...])
        out_ref[...] = hn_new
        if emit_ab:
            a_ref[...] = _dot(hn_new, wa_ref[...])
            b_ref[...] = _dot(hn_new, wb_ref[...])

    in_specs = [_w((r, 128))] * 3 + [_w((128, 128)), _w((128, 128)),
                                     _w((1, 128)), _w((128, 128)),
                                     _w((1, 128)), _w((1, 128)), _w((1, 128)),
                                     _w((128, 128))]
    args = [hn_p, p0_p, p1_p, _kron4(mp["w1"][0:32]), _kron4(mp["w1"][32:64]),
            _t4(mp["b1"]), _kron4(mp["w2"]), _t4(mp["b2"]),
            _t4(lnp["g"]), _t4(lnp["b"]), _mavg()]
    if emit_ab:
        in_specs += [_w((128, 128)), _w((128, 128))]
        args += [_kron4(wa), _kron4(wb)]
        out_specs = [_w((r, 128))] * 3
        out_shape = [jax.ShapeDtypeStruct((r, 128), jnp.float32)] * 3
    else:
        out_specs = _w((r, 128))
        out_shape = jax.ShapeDtypeStruct((r, 128), jnp.float32)
    return pl.pallas_call(
        body, grid=(1,), in_specs=in_specs, out_specs=out_specs,
        out_shape=out_shape)(*args)


def _tc_decoder(hn_p, m_p, mp):
    r = hn_p.shape[0]

    def body(hn_ref, m_ref, w1_ref, b1_ref, w2_ref, b2_ref, out_ref):
        h = jnp.maximum(_dot(hn_ref[...], w1_ref[...]) + b1_ref[...], 0.0)
        y = _dot(h, w2_ref[...]) + b2_ref[...]
        out_ref[...] = y * m_ref[...]

    return pl.pallas_call(
        body,
        grid=(1,),
        in_specs=[_w((r, 128)), _w((r, 12)), _w((128, 128)), _w((1, 128)),
                  _w((128, 12)), _w((1, 12))],
        out_specs=_w((r, 12)),
        out_shape=jax.ShapeDtypeStruct((r, 12), jnp.float32),
    )(hn_p, m_p, _kron4(mp["w1"]), _t4(mp["b1"]), _kron4(mp["w2"]),
      _t4(mp["b2"]))


@functools.cache
def _mesh():
    return plsc.VectorSubcoreMesh(core_axis_name="c", subcore_axis_name="s")


_NOTILE = pltpu.CompilerParams(use_tc_tiling_on_sc=False)
_CHR = 8
_CH = _CHR * 128


def _sc_gather(a, b, src2, dst2):
    e = src2.shape[0] * 128
    nch = e // _CH
    tail = (e - nch * _CH) // 128

    @functools.partial(
        pl.kernel,
        mesh=_mesh(),
        out_type=[jax.ShapeDtypeStruct((e, 32), jnp.float32),
                  jax.ShapeDtypeStruct((e, 32), jnp.float32)],
        scratch_types=[pltpu.VMEM((_CHR, 128), jnp.int32),
                       pltpu.VMEM((_CHR, 128), jnp.int32),
                       pltpu.VMEM((_CH, 32), jnp.float32),
                       pltpu.VMEM((_CH, 32), jnp.float32),
                       pltpu.SemaphoreType.DMA,
                       pltpu.SemaphoreType.DMA],
        compiler_params=_NOTILE,
    )
    def k(a_hbm, b_hbm, s_hbm, d_hbm, ga_hbm, gb_hbm, si, di, ba, bb, sa, sb):
        c = lax.axis_index("c")
        s = lax.axis_index("s")
        wid = c * 16 + s
        trips = (nch - wid + 31) // 32

        def do_chunk(ch, rows):
            pltpu.sync_copy(s_hbm.at[pl.ds(ch * _CHR, rows)],
                            si.at[pl.ds(0, rows)])
            pltpu.sync_copy(d_hbm.at[pl.ds(ch * _CHR, rows)],
                            di.at[pl.ds(0, rows)])
            cps = []
            for j in range(rows):
                cps.append(pltpu.async_copy(
                    a_hbm.at[si.at[j]], ba.at[pl.ds(j * 128, 128)], sa))
                cps.append(pltpu.async_copy(
                    b_hbm.at[di.at[j]], bb.at[pl.ds(j * 128, 128)], sb))
            for cp in cps:
                cp.wait()
            pltpu.sync_copy(ba.at[pl.ds(0, rows * 128)],
                            ga_hbm.at[pl.ds(ch * _CH, rows * 128)])
            pltpu.sync_copy(bb.at[pl.ds(0, rows * 128)],
                            gb_hbm.at[pl.ds(ch * _CH, rows * 128)])

        def body(i, carry):
            do_chunk(wid + i * 32, _CHR)
            return carry

        lax.fori_loop(0, trips, body, 0)
        if tail:
            @pl.when(wid == 31)
            def _():
                do_chunk(nch, tail)

    return k(a, b, src2, dst2)


def _sc_scatter(he, dst2, zeros):
    n = zeros.shape[0]
    e = he.shape[0]
    nch = e // _CH
    tail = (e - nch * _CH) // 128
    per = n // 16

    @functools.partial(
        pl.kernel,
        mesh=_mesh(),
        out_type=[jax.ShapeDtypeStruct((n, 32), jnp.float32),
                  jax.ShapeDtypeStruct((n, 32), jnp.float32)],
        scratch_types=[pltpu.VMEM((_CHR, 128), jnp.int32),
                       pltpu.VMEM((_CH, 32), jnp.float32),
                       pltpu.VMEM_SHARED((n, 32), jnp.float32)],
        compiler_params=_NOTILE,
    )
    def k(he_hbm, d_hbm, z_hbm, o0, o1, di, be, acc):
        c = lax.axis_index("c")
        s = lax.axis_index("s")
        wid = c * 16 + s
        pltpu.sync_copy(z_hbm.at[pl.ds(s * per, per)], acc.at[pl.ds(s * per, per)])
        plsc.subcore_barrier()
        trips = (nch - wid + 31) // 32

        def do_chunk(ch, rows):
            pltpu.sync_copy(d_hbm.at[pl.ds(ch * _CHR, rows)],
                            di.at[pl.ds(0, rows)])
            pltpu.sync_copy(he_hbm.at[pl.ds(ch * _CH, rows * 128)],
                            be.at[pl.ds(0, rows * 128)])
            for j in range(rows):
                pltpu.sync_copy(be.at[pl.ds(j * 128, 128)], acc.at[di.at[j]],
                                add=True)

        def body(i, carry):
            do_chunk(wid + i * 32, _CHR)
            return carry

        lax.fori_loop(0, trips, body, 0)
        if tail:
            @pl.when(wid == 30)
            def _():
                do_chunk(nch, tail)
        plsc.subcore_barrier()

        @pl.when(c == 0)
        def _():
            pltpu.sync_copy(acc.at[pl.ds(s * per, per)], o0.at[pl.ds(s * per, per)])

        @pl.when(c == 1)
        def _():
            pltpu.sync_copy(acc.at[pl.ds(s * per, per)], o1.at[pl.ds(s * per, per)])

    return k(he, dst2, zeros)


def kernel(x, edge_attr, params, edge_index):
    n = x.shape[0]
    e = edge_index.shape[1]
    src2 = edge_index[0].reshape(-1, 128)
    dst2 = edge_index[1].reshape(-1, 128)
    layers = params["layers"]
    ew = [lp["edge"]["w1"] for lp in layers]

    hn, a, b, m_p = _tc_node_encoder(x.reshape(n // 4, 512), params["enc_n"],
                                     params["enc_n_ln"],
                                     ew[0][32:64], ew[0][64:96])
    he = _tc_edge_encoder(edge_attr.reshape(e // 64, 128), params["enc_e"],
                          params["enc_e_ln"])
    zeros = jnp.zeros((n, 32), jnp.float32)

    for l, lp in enumerate(layers):
        ga, gb = _sc_gather(a.reshape(n, 32), b.reshape(n, 32), src2, dst2)
        he = _tc_edge_update(he, ga.reshape(e // 4, 128),
                             gb.reshape(e // 4, 128), lp["edge"],
                             lp["edge_ln"])
        p0, p1 = _sc_scatter(he.reshape(e, 32), dst2, zeros)
        if l + 1 < len(layers):
            hn, a, b = _tc_node_update(hn, p0.reshape(n // 4, 128),
                                       p1.reshape(n // 4, 128), lp["node"],
                                       lp["node_ln"],
                                       ew[l + 1][32:64], ew[l + 1][64:96])
        else:
            hn = _tc_node_update(hn, p0.reshape(n // 4, 128),
                                 p1.reshape(n // 4, 128), lp["node"],
                                 lp["node_ln"])

    return _tc_decoder(hn, m_p, params["dec"]).reshape(n, 3)
